# baseline (speedup 1.0000x reference)
import functools

import jax
import jax.numpy as jnp
from jax import lax
from jax.experimental import pallas as pl
from jax.experimental.pallas import tpu as pltpu
from jax.experimental.pallas import tpu_sc as plsc

_R, _C = 16384, 128
_NC, _NS, _L = 2, 16, 16
_NW = _NC * _NS
_RW = _R // _NW
_NCH = 8
_CH = _RW // _NCH

_mesh = plsc.VectorSubcoreMesh(core_axis_name="c", subcore_axis_name="s")


@functools.partial(
    pl.kernel,
    mesh=_mesh,
    out_type=jax.ShapeDtypeStruct((_R, _C), jnp.float32),
    scratch_types=[pltpu.VMEM((_NCH * _CH, _C), jnp.float32)]
    + [pltpu.SemaphoreType.DMA] * (2 * _NCH),
    compiler_params=pltpu.CompilerParams(needs_layout_passes=False),
)
def _sc_kernel(x_hbm, out_hbm, buf, *sems):
    sin = sems[:_NCH]
    sout = sems[_NCH:]
    wid = lax.axis_index("s") * _NC + lax.axis_index("c")
    base = wid * _RW

    in_h = [
        pltpu.async_copy(
            x_hbm.at[pl.ds(base + i * _CH, _CH)],
            buf.at[pl.ds(i * _CH, _CH)],
            sin[i],
        )
        for i in range(_NCH)
    ]

    lanes = lax.iota(jnp.int32, 16)
    c124 = jnp.full((16,), 124, jnp.int32)
    c125 = jnp.full((16,), 125, jnp.int32)
    c126 = jnp.full((16,), 126, jnp.int32)
    c127 = jnp.full((16,), 127, jnp.int32)

    out_h = []
    for i in range(_NCH):
        in_h[i].wait()

        def _group(g, _):
            rows = lanes + (i * _CH + g * _L)
            b = plsc.load_gather(buf, [rows, c124])
            cvb = plsc.load_gather(buf, [rows, c125])
            cvl = plsc.load_gather(buf, [rows, c126])
            qg = plsc.load_gather(buf, [rows, c127])
            c = b / (1.0 / cvb - 1.0)
            d = c / cvl - c
            plsc.store_scatter(buf, [rows, c125], c)
            plsc.store_scatter(buf, [rows, c126], (1.0 - qg) * d)
            plsc.store_scatter(buf, [rows, c127], qg * d)
            return 0

        lax.fori_loop(0, _CH // _L, _group, 0)
        out_h.append(
            pltpu.async_copy(
                buf.at[pl.ds(i * _CH, _CH)],
                out_hbm.at[pl.ds(base + i * _CH, _CH)],
                sout[i],
            )
        )
    for h in out_h:
        h.wait()


def kernel(x):
    return _sc_kernel(x)


# speedup vs baseline: 1.0014x; 1.0014x over previous
import functools

import jax
import jax.numpy as jnp
from jax import lax
from jax.experimental import pallas as pl
from jax.experimental.pallas import tpu as pltpu
from jax.experimental.pallas import tpu_sc as plsc

_R, _C = 16384, 128
_NC, _NS, _L = 2, 16, 16
_NW = _NC * _NS
_RW = _R // _NW
_NCH = 4
_CH = _RW // _NCH

_mesh = plsc.VectorSubcoreMesh(core_axis_name="c", subcore_axis_name="s")


@functools.partial(
    pl.kernel,
    mesh=_mesh,
    out_type=jax.ShapeDtypeStruct((_R, _C), jnp.float32),
    scratch_types=[pltpu.VMEM((_NCH * _CH, _C), jnp.float32)]
    + [pltpu.SemaphoreType.DMA] * (2 * _NCH),
    compiler_params=pltpu.CompilerParams(needs_layout_passes=False),
)
def _sc_kernel(x_hbm, out_hbm, buf, *sems):
    sin = sems[:_NCH]
    sout = sems[_NCH:]
    wid = lax.axis_index("s") * _NC + lax.axis_index("c")
    base = wid * _RW

    in_h = [
        pltpu.async_copy(
            x_hbm.at[pl.ds(base + i * _CH, _CH)],
            buf.at[pl.ds(i * _CH, _CH)],
            sin[i],
        )
        for i in range(_NCH)
    ]

    lanes = lax.iota(jnp.int32, 16)
    c124 = jnp.full((16,), 124, jnp.int32)
    c125 = jnp.full((16,), 125, jnp.int32)
    c126 = jnp.full((16,), 126, jnp.int32)
    c127 = jnp.full((16,), 127, jnp.int32)

    out_h = []
    for i in range(_NCH):
        in_h[i].wait()

        def _group(g, _):
            rows = lanes + (i * _CH + g * _L)
            b = plsc.load_gather(buf, [rows, c124])
            cvb = plsc.load_gather(buf, [rows, c125])
            cvl = plsc.load_gather(buf, [rows, c126])
            qg = plsc.load_gather(buf, [rows, c127])
            c = b / (1.0 / cvb - 1.0)
            d = c / cvl - c
            plsc.store_scatter(buf, [rows, c125], c)
            plsc.store_scatter(buf, [rows, c126], (1.0 - qg) * d)
            plsc.store_scatter(buf, [rows, c127], qg * d)
            return 0

        lax.fori_loop(0, _CH // _L, _group, 0)
        out_h.append(
            pltpu.async_copy(
                buf.at[pl.ds(i * _CH, _CH)],
                out_hbm.at[pl.ds(base + i * _CH, _CH)],
                sout[i],
            )
        )
    for h in out_h:
        h.wait()


def kernel(x):
    return _sc_kernel(x)
